# Initial kernel scaffold; baseline (speedup 1.0000x reference)
import functools

import jax
import jax.numpy as jnp
from jax import lax
from jax.experimental import pallas as pl
from jax.experimental.pallas import tpu as pltpu
from jax.experimental.pallas import tpu_sc as plsc

_B = 4096
_IN_DIM = 512
_OUT_DIM = 256
_E = 8192
_S_SIZE = _IN_DIM * _OUT_DIM

_NC = 2
_NS = 16
_NW = _NC * _NS
_EPT = _E // _NW
_ZCH = _S_SIZE // _NS
_LANES = 16


def _sc_scatter_body(cols_hbm, rows_hbm, w_hbm, zeros_hbm, out_hbm,
                     s_sh, idx_a, idx_b, w_a, w_b, cols_v, rows_v):
    c = lax.axis_index("c")
    s = lax.axis_index("s")
    wid = s * _NC + c
    base = wid * _EPT

    zsl = pl.ds(s * _ZCH, _ZCH)
    pltpu.sync_copy(zeros_hbm.at[zsl], s_sh.at[zsl])

    pltpu.sync_copy(cols_hbm.at[pl.ds(base, _EPT)], cols_v)
    pltpu.sync_copy(rows_hbm.at[pl.ds(base, _EPT)], rows_v)
    pltpu.sync_copy(w_hbm.at[pl.ds(base, 128)], w_a)
    pltpu.sync_copy(w_hbm.at[pl.ds(base + 128, 128)], w_b)

    for i in range(128 // _LANES):
        sl = pl.ds(i * _LANES, _LANES)
        idx_a[sl] = cols_v[sl] * _OUT_DIM + rows_v[sl]
    for i in range(128 // _LANES):
        src = pl.ds(128 + i * _LANES, _LANES)
        idx_b[pl.ds(i * _LANES, _LANES)] = cols_v[src] * _OUT_DIM + rows_v[src]

    plsc.subcore_barrier()
    pltpu.sync_copy(w_a, s_sh.at[idx_a], add=True)
    pltpu.sync_copy(w_b, s_sh.at[idx_b], add=True)
    plsc.subcore_barrier()

    pltpu.sync_copy(s_sh.at[zsl], out_hbm.at[c].at[zsl])


_sc_scatter = pl.kernel(
    _sc_scatter_body,
    out_type=jax.ShapeDtypeStruct((_NC, _S_SIZE), jnp.float32),
    mesh=plsc.VectorSubcoreMesh(core_axis_name="c", subcore_axis_name="s"),
    scratch_types=[
        pltpu.VMEM_SHARED((_S_SIZE,), jnp.float32),
        pltpu.VMEM((128,), jnp.int32),
        pltpu.VMEM((128,), jnp.int32),
        pltpu.VMEM((128,), jnp.float32),
        pltpu.VMEM((128,), jnp.float32),
        pltpu.VMEM((_EPT,), jnp.int32),
        pltpu.VMEM((_EPT,), jnp.int32),
    ],
)


_BLK = 512


def _tc_body(x_ref, s0_ref, s1_ref, act_ref, o_ref):
    smat = s0_ref[...] + s1_ref[...]
    pre = jnp.dot(x_ref[...], smat, preferred_element_type=jnp.float32)
    y = jnp.where(act_ref[...] == 1, jnp.tanh(pre), pre)
    o_ref[...] = jax.nn.sigmoid(y)


@jax.jit
def kernel(x, W, conn_rows, conn_cols, act_diag):
    cols = conn_cols.astype(jnp.int32)
    rows = conn_rows.astype(jnp.int32)
    w32 = W.astype(jnp.float32)
    zeros = jnp.zeros((_S_SIZE,), jnp.float32)

    parts = _sc_scatter(cols, rows, w32, zeros)
    s0 = parts[0].reshape(_IN_DIM, _OUT_DIM)
    s1 = parts[1].reshape(_IN_DIM, _OUT_DIM)
    act2 = act_diag.reshape(1, _OUT_DIM)

    return pl.pallas_call(
        _tc_body,
        grid=(_B // _BLK,),
        in_specs=[
            pl.BlockSpec((_BLK, _IN_DIM), lambda i: (i, 0)),
            pl.BlockSpec((_IN_DIM, _OUT_DIM), lambda i: (0, 0)),
            pl.BlockSpec((_IN_DIM, _OUT_DIM), lambda i: (0, 0)),
            pl.BlockSpec((1, _OUT_DIM), lambda i: (0, 0)),
        ],
        out_specs=pl.BlockSpec((_BLK, _OUT_DIM), lambda i: (i, 0)),
        out_shape=jax.ShapeDtypeStruct((_B, _OUT_DIM), jnp.float32),
    )(x, s0, s1, act2)


# speedup vs baseline: 8.1678x; 8.1678x over previous
import functools

import jax
import jax.numpy as jnp
from jax import lax
from jax.experimental import pallas as pl
from jax.experimental.pallas import tpu as pltpu
from jax.experimental.pallas import tpu_sc as plsc

_B = 4096
_IN_DIM = 512
_OUT_DIM = 256
_E = 8192
_S_SIZE = _IN_DIM * _OUT_DIM

_NC = 2
_NS = 16
_NW = _NC * _NS
_EPT = _E // _NW
_ZCH = _S_SIZE // _NS
_LANES = 16


def _sc_scatter_body(cols_hbm, rows_hbm, w_hbm, zeros_hbm, out_hbm,
                     s_sh, idx_a, idx_b, w_a, w_b, cols_v, rows_v):
    c = lax.axis_index("c")
    s = lax.axis_index("s")
    wid = s * _NC + c
    base = wid * _EPT

    zsl = pl.ds(s * _ZCH, _ZCH)
    pltpu.sync_copy(zeros_hbm.at[zsl], s_sh.at[zsl])

    pltpu.sync_copy(cols_hbm.at[pl.ds(base, _EPT)], cols_v)
    pltpu.sync_copy(rows_hbm.at[pl.ds(base, _EPT)], rows_v)
    pltpu.sync_copy(w_hbm.at[pl.ds(base, 128)], w_a)
    pltpu.sync_copy(w_hbm.at[pl.ds(base + 128, 128)], w_b)

    for i in range(128 // _LANES):
        sl = pl.ds(i * _LANES, _LANES)
        idx_a[sl] = cols_v[sl] * _OUT_DIM + rows_v[sl]
    for i in range(128 // _LANES):
        src = pl.ds(128 + i * _LANES, _LANES)
        idx_b[pl.ds(i * _LANES, _LANES)] = cols_v[src] * _OUT_DIM + rows_v[src]

    plsc.subcore_barrier()
    pltpu.sync_copy(w_a, s_sh.at[idx_a], add=True)
    pltpu.sync_copy(w_b, s_sh.at[idx_b], add=True)
    plsc.subcore_barrier()

    pltpu.sync_copy(s_sh.at[zsl], out_hbm.at[c].at[zsl])


@functools.cache
def _sc_scatter():
    return pl.kernel(
        _sc_scatter_body,
        out_type=jax.ShapeDtypeStruct((_NC, _S_SIZE), jnp.float32),
        mesh=plsc.VectorSubcoreMesh(core_axis_name="c", subcore_axis_name="s"),
        scratch_types=[
            pltpu.VMEM_SHARED((_S_SIZE,), jnp.float32),
            pltpu.VMEM((128,), jnp.int32),
            pltpu.VMEM((128,), jnp.int32),
            pltpu.VMEM((128,), jnp.float32),
            pltpu.VMEM((128,), jnp.float32),
            pltpu.VMEM((_EPT,), jnp.int32),
            pltpu.VMEM((_EPT,), jnp.int32),
        ],
    )


_BLK = 512


def _tc_body(x_ref, s0_ref, s1_ref, act_ref, o_ref):
    smat = s0_ref[...] + s1_ref[...]
    pre = jnp.dot(x_ref[...], smat, preferred_element_type=jnp.float32)
    y = jnp.where(act_ref[...] == 1, jnp.tanh(pre), pre)
    o_ref[...] = jax.nn.sigmoid(y)


@jax.jit
def kernel(x, W, conn_rows, conn_cols, act_diag):
    cols = conn_cols.astype(jnp.int32)
    rows = conn_rows.astype(jnp.int32)
    w32 = W.astype(jnp.float32)
    zeros = jnp.zeros((_S_SIZE,), jnp.float32)

    parts = _sc_scatter()(cols, rows, w32, zeros)
    s0 = parts[0].reshape(_IN_DIM, _OUT_DIM)
    s1 = parts[1].reshape(_IN_DIM, _OUT_DIM)
    act2 = act_diag.reshape(1, _OUT_DIM)

    return pl.pallas_call(
        _tc_body,
        grid=(_B // _BLK,),
        in_specs=[
            pl.BlockSpec((_BLK, _IN_DIM), lambda i: (i, 0)),
            pl.BlockSpec((_IN_DIM, _OUT_DIM), lambda i: (0, 0)),
            pl.BlockSpec((_IN_DIM, _OUT_DIM), lambda i: (0, 0)),
            pl.BlockSpec((1, _OUT_DIM), lambda i: (0, 0)),
        ],
        out_specs=pl.BlockSpec((_BLK, _OUT_DIM), lambda i: (i, 0)),
        out_shape=jax.ShapeDtypeStruct((_B, _OUT_DIM), jnp.float32),
    )(x, s0, s1, act2)


# speedup vs baseline: 8.6951x; 1.0645x over previous
import functools

import jax
import jax.numpy as jnp
from jax import lax
from jax.experimental import pallas as pl
from jax.experimental.pallas import tpu as pltpu
from jax.experimental.pallas import tpu_sc as plsc

_B = 4096
_IN_DIM = 512
_OUT_DIM = 256
_E = 8192
_S_SIZE = _IN_DIM * _OUT_DIM

_NC = 2
_NS = 16
_HALF = _S_SIZE // _NC
_ACC = _HALF + 16
_EPT = _E // _NS
_ZCH = _HALF // _NS
_LANES = 16
_NSEG = _EPT // 128


def _sc_scatter_body(cols_hbm, rows_hbm, w_hbm, zeros_hbm, out_hbm,
                     acc, cols_v, rows_v, idx_segs, w_segs):
    c = lax.axis_index("c")
    s = lax.axis_index("s")
    base = s * _EPT

    zsl = pl.ds(s * _ZCH, _ZCH)
    pltpu.sync_copy(zeros_hbm.at[zsl], acc.at[zsl])

    pltpu.sync_copy(cols_hbm.at[pl.ds(base, _EPT)], cols_v)
    pltpu.sync_copy(rows_hbm.at[pl.ds(base, _EPT)], rows_v)
    for j in range(_NSEG):
        pltpu.sync_copy(w_hbm.at[pl.ds(base + j * 128, 128)], w_segs.at[j])

    lo = c * _HALF
    for j in range(_NSEG):
        for i in range(128 // _LANES):
            sl = pl.ds(j * 128 + i * _LANES, _LANES)
            local = cols_v[sl] * _OUT_DIM + rows_v[sl] - lo
            inr = (local >= 0) & (local < _HALF)
            idx_segs[j, pl.ds(i * _LANES, _LANES)] = jnp.where(
                inr, local, _HALF)

    plsc.subcore_barrier()
    for j in range(_NSEG):
        pltpu.sync_copy(w_segs.at[j], acc.at[idx_segs.at[j]], add=True)
    plsc.subcore_barrier()

    pltpu.sync_copy(acc.at[zsl], out_hbm.at[pl.ds(c * _HALF + s * _ZCH, _ZCH)])


@functools.cache
def _sc_scatter():
    return pl.kernel(
        _sc_scatter_body,
        out_type=jax.ShapeDtypeStruct((_S_SIZE,), jnp.float32),
        mesh=plsc.VectorSubcoreMesh(core_axis_name="c", subcore_axis_name="s"),
        scratch_types=[
            pltpu.VMEM_SHARED((_ACC,), jnp.float32),
            pltpu.VMEM((_EPT,), jnp.int32),
            pltpu.VMEM((_EPT,), jnp.int32),
            pltpu.VMEM((_NSEG, 128), jnp.int32),
            pltpu.VMEM((_NSEG, 128), jnp.float32),
        ],
    )


_BLK = 1024


def _tc_body(x_ref, s_ref, act_ref, o_ref):
    pre = jnp.dot(x_ref[...], s_ref[...], preferred_element_type=jnp.float32)
    y = jnp.where(act_ref[...] == 1, jnp.tanh(pre), pre)
    o_ref[...] = jax.nn.sigmoid(y)


@jax.jit
def kernel(x, W, conn_rows, conn_cols, act_diag):
    cols = conn_cols.astype(jnp.int32)
    rows = conn_rows.astype(jnp.int32)
    w32 = W.astype(jnp.float32)
    zeros = jnp.zeros((_HALF,), jnp.float32)

    s_flat = _sc_scatter()(cols, rows, w32, zeros)
    smat = s_flat.reshape(_IN_DIM, _OUT_DIM)
    act2 = act_diag.reshape(1, _OUT_DIM)

    return pl.pallas_call(
        _tc_body,
        grid=(_B // _BLK,),
        in_specs=[
            pl.BlockSpec((_BLK, _IN_DIM), lambda i: (i, 0)),
            pl.BlockSpec((_IN_DIM, _OUT_DIM), lambda i: (0, 0)),
            pl.BlockSpec((1, _OUT_DIM), lambda i: (0, 0)),
        ],
        out_specs=pl.BlockSpec((_BLK, _OUT_DIM), lambda i: (i, 0)),
        out_shape=jax.ShapeDtypeStruct((_B, _OUT_DIM), jnp.float32),
    )(x, smat, act2)


# speedup vs baseline: 9.8512x; 1.1330x over previous
import functools

import jax
import jax.numpy as jnp
from jax import lax
from jax.experimental import pallas as pl
from jax.experimental.pallas import tpu as pltpu
from jax.experimental.pallas import tpu_sc as plsc

_B = 4096
_IN_DIM = 512
_OUT_DIM = 256
_E = 8192
_S_SIZE = _IN_DIM * _OUT_DIM

_NC = 2
_NS = 16
_HALF = _S_SIZE // _NC
_ACC = _HALF + 16
_EPT = _E // _NS
_ZCH = _HALF // _NS
_LANES = 16
_NSEG = _EPT // 128


def _sc_scatter_body(cols_hbm, rows_hbm, w_hbm, out_hbm,
                     acc, cols_v, rows_v, idx_segs, w_segs, zbuf,
                     sem_in, sem_z, sem_sc):
    c = lax.axis_index("c")
    s = lax.axis_index("s")
    base = s * _EPT

    cp_in = [
        pltpu.async_copy(cols_hbm.at[pl.ds(base, _EPT)], cols_v, sem_in),
        pltpu.async_copy(rows_hbm.at[pl.ds(base, _EPT)], rows_v, sem_in),
    ]
    for j in range(_NSEG):
        cp_in.append(pltpu.async_copy(
            w_hbm.at[pl.ds(base + j * 128, 128)], w_segs.at[j], sem_in))

    zvec = jnp.zeros((_LANES,), jnp.float32)

    def _zbody(i, _):
        zbuf[pl.ds(i * _LANES, _LANES)] = zvec
        return 0

    lax.fori_loop(0, _ZCH // _LANES, _zbody, 0)
    zsl = pl.ds(s * _ZCH, _ZCH)
    cp_z = pltpu.async_copy(zbuf, acc.at[zsl], sem_z)

    for cp in cp_in:
        cp.wait()

    for j in range(_NSEG):
        for i in range(128 // _LANES):
            sl = pl.ds(j * 128 + i * _LANES, _LANES)
            row = rows_v[sl]
            p = ((row >> 7) - c) * _HALF + cols_v[sl] * 128 + (row & 127)
            inr = (p >= 0) & (p < _HALF)
            idx_segs[j, pl.ds(i * _LANES, _LANES)] = jnp.where(inr, p, _HALF)

    cp_z.wait()
    plsc.subcore_barrier()
    cp_sc = [
        pltpu.async_copy(w_segs.at[j], acc.at[idx_segs.at[j]], sem_sc,
                         add=True)
        for j in range(_NSEG)
    ]
    for cp in cp_sc:
        cp.wait()
    plsc.subcore_barrier()

    pltpu.sync_copy(acc.at[zsl], out_hbm.at[pl.ds(c * _HALF + s * _ZCH, _ZCH)])


@functools.cache
def _sc_scatter():
    return pl.kernel(
        _sc_scatter_body,
        out_type=jax.ShapeDtypeStruct((_S_SIZE,), jnp.float32),
        mesh=plsc.VectorSubcoreMesh(core_axis_name="c", subcore_axis_name="s"),
        scratch_types=[
            pltpu.VMEM_SHARED((_ACC,), jnp.float32),
            pltpu.VMEM((_EPT,), jnp.int32),
            pltpu.VMEM((_EPT,), jnp.int32),
            pltpu.VMEM((_NSEG, 128), jnp.int32),
            pltpu.VMEM((_NSEG, 128), jnp.float32),
            pltpu.VMEM((_ZCH,), jnp.float32),
            pltpu.SemaphoreType.DMA,
            pltpu.SemaphoreType.DMA,
            pltpu.SemaphoreType.DMA,
        ],
    )


_BLK = 1024
_HOUT = _OUT_DIM // 2


def _tc_body(x_ref, a_ref, act_ref, o_ref):
    xb = x_ref[...]
    left = a_ref[0:_IN_DIM, :]
    right = a_ref[_IN_DIM:2 * _IN_DIM, :]
    pre_l = jnp.dot(xb, left, preferred_element_type=jnp.float32)
    pre_r = jnp.dot(xb, right, preferred_element_type=jnp.float32)
    al = act_ref[:, 0:_HOUT]
    ar = act_ref[:, _HOUT:_OUT_DIM]
    yl = jnp.where(al == 1, jnp.tanh(pre_l), pre_l)
    yr = jnp.where(ar == 1, jnp.tanh(pre_r), pre_r)
    o_ref[:, 0:_HOUT] = jax.nn.sigmoid(yl)
    o_ref[:, _HOUT:_OUT_DIM] = jax.nn.sigmoid(yr)


@jax.jit
def kernel(x, W, conn_rows, conn_cols, act_diag):
    cols = conn_cols.astype(jnp.int32)
    rows = conn_rows.astype(jnp.int32)
    w32 = W.astype(jnp.float32)

    s_flat = _sc_scatter()(cols, rows, w32)
    amat = s_flat.reshape(2 * _IN_DIM, _HOUT)
    act2 = act_diag.reshape(1, _OUT_DIM)

    return pl.pallas_call(
        _tc_body,
        grid=(_B // _BLK,),
        in_specs=[
            pl.BlockSpec((_BLK, _IN_DIM), lambda i: (i, 0)),
            pl.BlockSpec((2 * _IN_DIM, _HOUT), lambda i: (0, 0)),
            pl.BlockSpec((1, _OUT_DIM), lambda i: (0, 0)),
        ],
        out_specs=pl.BlockSpec((_BLK, _OUT_DIM), lambda i: (i, 0)),
        out_shape=jax.ShapeDtypeStruct((_B, _OUT_DIM), jnp.float32),
    )(x, amat, act2)


# speedup vs baseline: 9.9920x; 1.0143x over previous
import functools

import jax
import jax.numpy as jnp
from jax import lax
from jax.experimental import pallas as pl
from jax.experimental.pallas import tpu as pltpu
from jax.experimental.pallas import tpu_sc as plsc

_B = 4096
_IN_DIM = 512
_OUT_DIM = 256
_E = 8192
_S_SIZE = _IN_DIM * _OUT_DIM

_NC = 2
_NS = 16
_HALF = _S_SIZE // _NC
_ACC = _HALF + 16
_EPT = _E // _NS
_ZCH = _HALF // _NS
_LANES = 16
_NSEG = _EPT // 128


def _sc_scatter_body(cols_hbm, rows_hbm, w_hbm, out_hbm,
                     acc, cols_v, rows_v, idx_segs, w_segs, zbuf,
                     sem_in, sem_z, sem_sc):
    c = lax.axis_index("c")
    s = lax.axis_index("s")
    base = s * _EPT

    cp_in = [
        pltpu.async_copy(cols_hbm.at[pl.ds(base, _EPT)], cols_v, sem_in),
        pltpu.async_copy(rows_hbm.at[pl.ds(base, _EPT)], rows_v, sem_in),
    ]
    for j in range(_NSEG):
        cp_in.append(pltpu.async_copy(
            w_hbm.at[pl.ds(base + j * 128, 128)], w_segs.at[j], sem_in))

    zvec = jnp.zeros((_LANES,), jnp.float32)
    _ZUNROLL = 16

    def _zbody(i, _):
        for u in range(_ZUNROLL):
            zbuf[pl.ds((i * _ZUNROLL + u) * _LANES, _LANES)] = zvec
        return 0

    lax.fori_loop(0, _ZCH // (_LANES * _ZUNROLL), _zbody, 0)
    zsl = pl.ds(s * _ZCH, _ZCH)
    cp_z = pltpu.async_copy(zbuf, acc.at[zsl], sem_z)

    for cp in cp_in:
        cp.wait()

    for j in range(_NSEG):
        for i in range(128 // _LANES):
            sl = pl.ds(j * 128 + i * _LANES, _LANES)
            row = rows_v[sl]
            p = ((row >> 7) - c) * _HALF + cols_v[sl] * 128 + (row & 127)
            inr = (p >= 0) & (p < _HALF)
            idx_segs[j, pl.ds(i * _LANES, _LANES)] = jnp.where(inr, p, _HALF)

    cp_z.wait()
    plsc.subcore_barrier()
    cp_sc = [
        pltpu.async_copy(w_segs.at[j], acc.at[idx_segs.at[j]], sem_sc,
                         add=True)
        for j in range(_NSEG)
    ]
    for cp in cp_sc:
        cp.wait()
    plsc.subcore_barrier()

    pltpu.sync_copy(acc.at[zsl], out_hbm.at[pl.ds(c * _HALF + s * _ZCH, _ZCH)])


@functools.cache
def _sc_scatter():
    return pl.kernel(
        _sc_scatter_body,
        out_type=jax.ShapeDtypeStruct((_S_SIZE,), jnp.float32),
        mesh=plsc.VectorSubcoreMesh(core_axis_name="c", subcore_axis_name="s"),
        scratch_types=[
            pltpu.VMEM_SHARED((_ACC,), jnp.float32),
            pltpu.VMEM((_EPT,), jnp.int32),
            pltpu.VMEM((_EPT,), jnp.int32),
            pltpu.VMEM((_NSEG, 128), jnp.int32),
            pltpu.VMEM((_NSEG, 128), jnp.float32),
            pltpu.VMEM((_ZCH,), jnp.float32),
            pltpu.SemaphoreType.DMA,
            pltpu.SemaphoreType.DMA,
            pltpu.SemaphoreType.DMA,
        ],
    )


_BLK = 1024
_HOUT = _OUT_DIM // 2


def _tc_body(x_ref, a_ref, act_ref, o_ref):
    xb = x_ref[...]
    left = a_ref[0:_IN_DIM, :]
    right = a_ref[_IN_DIM:2 * _IN_DIM, :]
    pre_l = jnp.dot(xb, left, preferred_element_type=jnp.float32)
    pre_r = jnp.dot(xb, right, preferred_element_type=jnp.float32)
    al = act_ref[:, 0:_HOUT]
    ar = act_ref[:, _HOUT:_OUT_DIM]
    yl = jnp.where(al == 1, jnp.tanh(pre_l), pre_l)
    yr = jnp.where(ar == 1, jnp.tanh(pre_r), pre_r)
    o_ref[:, 0:_HOUT] = jax.nn.sigmoid(yl)
    o_ref[:, _HOUT:_OUT_DIM] = jax.nn.sigmoid(yr)


@jax.jit
def kernel(x, W, conn_rows, conn_cols, act_diag):
    cols = conn_cols.astype(jnp.int32)
    rows = conn_rows.astype(jnp.int32)
    w32 = W.astype(jnp.float32)

    s_flat = _sc_scatter()(cols, rows, w32)
    amat = s_flat.reshape(2 * _IN_DIM, _HOUT)
    act2 = act_diag.reshape(1, _OUT_DIM)

    return pl.pallas_call(
        _tc_body,
        grid=(_B // _BLK,),
        in_specs=[
            pl.BlockSpec((_BLK, _IN_DIM), lambda i: (i, 0)),
            pl.BlockSpec((2 * _IN_DIM, _HOUT), lambda i: (0, 0)),
            pl.BlockSpec((1, _OUT_DIM), lambda i: (0, 0)),
        ],
        out_specs=pl.BlockSpec((_BLK, _OUT_DIM), lambda i: (i, 0)),
        out_shape=jax.ShapeDtypeStruct((_B, _OUT_DIM), jnp.float32),
    )(x, amat, act2)
